# baseline (speedup 1.0000x reference)
import functools

import jax
import jax.numpy as jnp
from jax import lax
from jax.experimental import pallas as pl

R = 50000
U = 10000
I = 5000
S = 7
M = 10000
CLASS = 2


def _final_block(him_ref, hrm_ref, hum_ref, pe_ref, lab_ref, w_ref,
                 loss_ref, cnt_ref):
    him = him_ref[...]
    hrm = hrm_ref[...]
    hum = hum_ref[...]
    pe_raw = pe_ref[...]
    lab = lab_ref[...]
    w = w_ref[...]
    pe = jnp.maximum(pe_raw, 0.0)
    sq = jnp.sum(pe * pe, axis=1, keepdims=True)
    pe = pe * lax.rsqrt(jnp.maximum(sq, 1e-12))
    raw = (jnp.dot(him, w[0:128, :], preferred_element_type=jnp.float32)
           + jnp.dot(hrm, w[128:192, :], preferred_element_type=jnp.float32)
           + jnp.dot(hum, w[192:320, :], preferred_element_type=jnp.float32)
           + jnp.dot(pe, w[320:384, :], preferred_element_type=jnp.float32))
    mx = jnp.max(raw, axis=1, keepdims=True)
    e = jnp.exp(raw - mx)
    p = e / jnp.sum(e, axis=1, keepdims=True)
    z = lab * p
    loss = jnp.sum(jnp.log1p(jnp.exp(-z)))
    pred1 = raw[:, 1] > raw[:, 0]
    truth1 = lab[:, 1] > lab[:, 0]
    correct = jnp.sum((pred1 == truth1).astype(jnp.float32))
    loss_ref[...] = jnp.reshape(loss, (1, 1))
    cnt_ref[...] = jnp.reshape(correct, (1, 1))


def _final(him, hrm, hum, pe_raw, lab, w):
    return pl.pallas_call(
        _final_block,
        out_shape=(jax.ShapeDtypeStruct((1, 1), jnp.float32),
                   jax.ShapeDtypeStruct((1, 1), jnp.float32)),
    )(him, hrm, hum, pe_raw, lab, w)


def kernel(review_feat, user_feat, item_feat, r_feature, label,
           u_review_adj, u_item_adj, i_review_adj, i_user_adj,
           r_user_adj, r_item_adj, r_sup_src, r_sup_dst, r_sup_val,
           idx_mask, W_r_agg, W_user, W_item, W_gcn, u_weight):
    idx = idx_mask
    ru_idx = jnp.take(r_user_adj, idx, axis=0)
    ri_idx = jnp.take(r_item_adj, idx, axis=0)

    h_r_in = jnp.concatenate([jnp.take(review_feat, idx, axis=0),
                              jnp.take(user_feat, ru_idx, axis=0),
                              jnp.take(item_feat, ri_idx, axis=0)], axis=1)
    h_r_m = jax.nn.relu(h_r_in @ W_r_agg)

    ur = jnp.take(review_feat, u_review_adj, axis=0)
    riu = jnp.take(item_feat, u_item_adj, axis=0)
    cu = jnp.concatenate([ur, riu], axis=2).reshape(U, -1)
    h_u = jnp.concatenate([user_feat, jax.nn.relu(cu @ W_user)], axis=1)
    ir = jnp.take(review_feat, i_review_adj, axis=0)
    rui = jnp.take(user_feat, i_user_adj, axis=0)
    ci = jnp.concatenate([ir, rui], axis=2).reshape(I, -1)
    h_i = jnp.concatenate([item_feat, jax.nn.relu(ci @ W_item)], axis=1)

    xw = r_feature @ W_gcn
    pos = jnp.full((R,), -1, jnp.int32).at[idx].set(
        jnp.arange(M, dtype=jnp.int32))
    pdst = jnp.take(pos, r_sup_dst, axis=0)
    dst2 = jnp.where(pdst >= 0, pdst, M)
    msgs = jnp.take(xw, r_sup_src, axis=0) * r_sup_val[:, None]
    acc = jax.ops.segment_sum(msgs, dst2, num_segments=M + 1)[:M]
    pe_m = jnp.take(acc, jnp.take(pos, idx, axis=0), axis=0)

    him_g = jnp.take(h_i, ri_idx, axis=0)
    hum_g = jnp.take(h_u, ru_idx, axis=0)
    lab_m = jnp.take(label, idx, axis=0)

    loss, cnt = _final(him_g, h_r_m, hum_g, pe_m, lab_m, u_weight)
    return (loss[0, 0], cnt[0, 0] / M)


# speedup vs baseline: 3.6527x; 3.6527x over previous
import functools

import jax
import jax.numpy as jnp
from jax import lax
from jax.experimental import pallas as pl
from jax.experimental.pallas import tpu as pltpu
from jax.experimental.pallas import tpu_sc as plsc

R = 50000
U = 10000
I = 5000
S = 7
M = 10000
CLASS = 2

NC = 2
NS = 16
NW = NC * NS


def _sc_mesh():
    return plsc.VectorSubcoreMesh(core_axis_name="c", subcore_axis_name="s")


@functools.cache
def _make_gather(B, D, dtype_name, CH):
    dtype = jnp.dtype(dtype_name)
    C = B // NW
    assert B % NW == 0 and C % CH == 0 and CH % 8 == 0
    nit = C // CH

    @functools.partial(
        pl.kernel,
        out_type=jax.ShapeDtypeStruct((B, D), dtype),
        mesh=_sc_mesh(),
        scratch_types=[pltpu.VMEM((CH,), jnp.int32),
                       pltpu.VMEM((CH, D), dtype),
                       pltpu.SemaphoreType.DMA],
        compiler_params=pltpu.CompilerParams(use_tc_tiling_on_sc=False,
                                              needs_layout_passes=False),
    )
    def g(table_hbm, idx_hbm, out_hbm, idx_v, rows_v, sem):
        wid = lax.axis_index("s") * NC + lax.axis_index("c")
        base = wid * C
        for j in range(nit):
            off = base + j * CH
            pltpu.sync_copy(idx_hbm.at[pl.ds(off, CH)], idx_v)
            pltpu.async_copy(table_hbm.at[idx_v], rows_v, sem).wait()
            pltpu.sync_copy(rows_v, out_hbm.at[pl.ds(off, CH)])

    return g


R_PAD = 50176
M_PAD = 10240
POS_C = R_PAD // NW
E_BLK = 512
E_W = 31744
E_PAD = E_W * NW
FIRE = 64
CBUF = 608
DUMMY = M


@functools.cache
def _make_pos_build():
    @functools.partial(
        pl.kernel,
        out_type=jax.ShapeDtypeStruct((R_PAD,), jnp.int32),
        mesh=_sc_mesh(),
        scratch_types=[pltpu.VMEM((M_PAD,), jnp.int32),
                       pltpu.VMEM((POS_C,), jnp.int32)],
        compiler_params=pltpu.CompilerParams(use_tc_tiling_on_sc=False,
                                              needs_layout_passes=False),
    )
    def pos_build(idx_hbm, pos_hbm, idx_v, pos_loc):
        wid = lax.axis_index("s") * NC + lax.axis_index("c")
        lo = wid * POS_C
        neg = jnp.full((16,), -1, jnp.int32)

        def fill(i, _):
            pos_loc[pl.ds(i * 16, 16)] = neg
            return 0
        lax.fori_loop(0, POS_C // 16, fill, 0)
        pltpu.sync_copy(idx_hbm, idx_v)
        iota = lax.iota(jnp.int32, 16)

        def scat(j, _):
            iv = idx_v[pl.ds(j * 16, 16)]
            mv = iota + j * 16
            msk = (iv >= lo) & (iv < lo + POS_C)
            plsc.store_scatter(pos_loc, [iv - lo], mv, mask=msk)
            return 0
        lax.fori_loop(0, M_PAD // 16, scat, 0)
        pltpu.sync_copy(pos_loc, pos_hbm.at[pl.ds(lo, POS_C)])

    return pos_build


@functools.cache
def _make_edge_pass():
    @functools.partial(
        pl.kernel,
        out_type=(jax.ShapeDtypeStruct((M_PAD, 64), jnp.float32),
                  jax.ShapeDtypeStruct((M_PAD, 64), jnp.float32)),
        mesh=_sc_mesh(),
        scratch_types=[pltpu.VMEM((R_PAD,), jnp.int32),
                       pltpu.VMEM((E_BLK,), jnp.int32),
                       pltpu.VMEM((E_BLK,), jnp.int32),
                       pltpu.VMEM((E_BLK,), jnp.float32),
                       pltpu.VMEM((CBUF,), jnp.int32),
                       pltpu.VMEM((CBUF,), jnp.int32),
                       pltpu.VMEM((CBUF,), jnp.float32),
                       pltpu.VMEM((FIRE,), jnp.int32),
                       pltpu.VMEM((FIRE,), jnp.int32),
                       pltpu.VMEM((FIRE, 64), jnp.float32),
                       pltpu.VMEM((128, 64), jnp.float32),
                       pltpu.VMEM_SHARED((M_PAD, 64), jnp.float32),
                       pltpu.SemaphoreType.DMA],
        compiler_params=pltpu.CompilerParams(use_tc_tiling_on_sc=False,
                                              needs_layout_passes=False),
    )
    def edge_pass(xw_hbm, src_hbm, dst_hbm, val_hbm, pos_hbm, out0, out1,
                  pos_tile, src_ch, dst_ch, val_ch, srcC, posC, valC,
                  srcF, posF, rows, zbuf, acc, sem):
        cid = lax.axis_index("c")
        sid = lax.axis_index("s")
        wid = sid * NC + cid
        rows_per_sub = M_PAD // NS
        pltpu.sync_copy(pos_hbm, pos_tile)
        zero16 = jnp.zeros((16,), jnp.float32)
        z16i = jnp.zeros((16,), jnp.int32)
        d16 = jnp.full((16,), DUMMY, jnp.int32)
        iota = lax.iota(jnp.int32, 16)

        def zf(i, _):
            for c in range(4):
                zbuf[i, pl.ds(c * 16, 16)] = zero16
            return 0
        lax.fori_loop(0, 128, zf, 0)
        for t in range(rows_per_sub // 128):
            pltpu.sync_copy(zbuf,
                            acc.at[pl.ds(sid * rows_per_sub + t * 128, 128)])

        def pf(j, _):
            srcC[pl.ds(j * 16, 16)] = z16i
            posC[pl.ds(j * 16, 16)] = d16
            valC[pl.ds(j * 16, 16)] = zero16
            return 0
        lax.fori_loop(0, CBUF // 16, pf, 0)
        plsc.subcore_barrier()

        base = wid * E_W

        def blk(b, _):
            off = base + b * E_BLK
            pltpu.sync_copy(src_hbm.at[pl.ds(off, E_BLK)], src_ch)
            pltpu.sync_copy(dst_hbm.at[pl.ds(off, E_BLK)], dst_ch)
            pltpu.sync_copy(val_hbm.at[pl.ds(off, E_BLK)], val_ch)

            def cmp(g, k):
                dv = dst_ch[pl.ds(g * 16, 16)]
                pv = plsc.load_gather(pos_tile, [dv])
                msk = pv >= 0
                sv = src_ch[pl.ds(g * 16, 16)]
                vv = val_ch[pl.ds(g * 16, 16)]
                plsc.store_compressed(srcC.at[pl.ds(k, 16)], sv, mask=msk)
                plsc.store_compressed(posC.at[pl.ds(k, 16)], pv, mask=msk)
                plsc.store_compressed(valC.at[pl.ds(k, 16)], vv, mask=msk)
                return k + jnp.sum(msk.astype(jnp.int32))
            k = lax.fori_loop(0, E_BLK // 16, cmp, jnp.int32(0))

            p0 = (k // 16) * 16
            for t in range(5):
                ids = iota + (p0 + t * 16)
                m2 = ids >= k
                plsc.store_scatter(srcC, [ids], z16i, mask=m2)
                plsc.store_scatter(posC, [ids], d16, mask=m2)
                plsc.store_scatter(valC, [ids], zero16, mask=m2)

            nf = (k + FIRE - 1) // FIRE

            def fire(t, _):
                o = t * FIRE
                for u in range(FIRE // 16):
                    srcF[pl.ds(u * 16, 16)] = srcC[pl.ds(o + u * 16, 16)]
                    posF[pl.ds(u * 16, 16)] = posC[pl.ds(o + u * 16, 16)]
                pltpu.async_copy(xw_hbm.at[srcF], rows, sem).wait()
                for g in range(FIRE // 16):
                    rid = iota + g * 16
                    valv = valC[pl.ds(o + g * 16, 16)]

                    def scale_f(f, _):
                        f16 = jnp.full((16,), f, jnp.int32)
                        colv = plsc.load_gather(rows, [rid, f16])
                        plsc.store_scatter(rows, [rid, f16], colv * valv)
                        return 0
                    lax.fori_loop(0, 64, scale_f, 0)
                pltpu.sync_copy(rows, acc.at[posF], add=True)
                return 0
            lax.fori_loop(0, nf, fire, 0)
            return 0
        lax.fori_loop(0, E_W // E_BLK, blk, 0)

        plsc.subcore_barrier()
        for t in range(rows_per_sub // 128):
            o = sid * rows_per_sub + t * 128
            pltpu.sync_copy(acc.at[pl.ds(o, 128)], zbuf)

            @pl.when(cid == 0)
            def _():
                pltpu.sync_copy(zbuf, out0.at[pl.ds(o, 128)])

            @pl.when(cid == 1)
            def _():
                pltpu.sync_copy(zbuf, out1.at[pl.ds(o, 128)])

    return edge_pass


@functools.cache
def _make_pe_assemble():
    C = M_PAD // NW

    @functools.partial(
        pl.kernel,
        out_type=jax.ShapeDtypeStruct((M_PAD, 64), jnp.float32),
        mesh=_sc_mesh(),
        scratch_types=[pltpu.VMEM((R_PAD,), jnp.int32),
                       pltpu.VMEM((C,), jnp.int32),
                       pltpu.VMEM((C,), jnp.int32),
                       pltpu.VMEM((C, 64), jnp.float32),
                       pltpu.VMEM((C, 64), jnp.float32),
                       pltpu.SemaphoreType.DMA],
        compiler_params=pltpu.CompilerParams(use_tc_tiling_on_sc=False,
                                              needs_layout_passes=False),
    )
    def pe_assemble(part0, part1, pos_hbm, idx_hbm, out,
                    pos_tile, idx_v, w_v, r0, r1, sem):
        wid = lax.axis_index("s") * NC + lax.axis_index("c")
        base = wid * C
        pltpu.sync_copy(pos_hbm, pos_tile)
        pltpu.sync_copy(idx_hbm.at[pl.ds(base, C)], idx_v)

        def wg(j, _):
            iv = idx_v[pl.ds(j * 16, 16)]
            w_v[pl.ds(j * 16, 16)] = plsc.load_gather(pos_tile, [iv])
            return 0
        lax.fori_loop(0, C // 16, wg, 0)
        pltpu.async_copy(part0.at[w_v], r0, sem).wait()
        pltpu.async_copy(part1.at[w_v], r1, sem).wait()

        def addr(i, _):
            for c in range(4):
                s = pl.ds(c * 16, 16)
                r0[i, s] = r0[i, s] + r1[i, s]
            return 0
        lax.fori_loop(0, C, addr, 0)
        pltpu.sync_copy(r0, out.at[pl.ds(base, C)])

    return pe_assemble


def _pad_idx(idx, B):
    return jnp.concatenate(
        [idx.astype(jnp.int32),
         jnp.zeros((B - idx.shape[0],), jnp.int32)])


def _sc_gather(table, idx, CH=320):
    n = idx.shape[0]
    B = -(-n // (NW * CH)) * (NW * CH)
    g = _make_gather(B, table.shape[1], str(table.dtype), CH)
    return g(table, _pad_idx(idx, B))[:n]


def _xw_block(x_ref, w_ref, o_ref):
    o_ref[...] = jnp.dot(x_ref[...], w_ref[...],
                         preferred_element_type=jnp.float32)


def _xw_matmul(x, w, blk=5000):
    n = x.shape[0]
    return pl.pallas_call(
        _xw_block,
        grid=(n // blk,),
        in_specs=[pl.BlockSpec((blk, x.shape[1]), lambda i: (i, 0)),
                  pl.BlockSpec(w.shape, lambda i: (0, 0))],
        out_specs=pl.BlockSpec((blk, w.shape[1]), lambda i: (i, 0)),
        out_shape=jax.ShapeDtypeStruct((n, w.shape[1]), jnp.float32),
    )(x, w)


def _hr_block(a_ref, b_ref, c_ref, w_ref, o_ref):
    w = w_ref[...]
    o_ref[...] = jnp.maximum(
        jnp.dot(a_ref[...], w[0:64], preferred_element_type=jnp.float32)
        + jnp.dot(b_ref[...], w[64:128], preferred_element_type=jnp.float32)
        + jnp.dot(c_ref[...], w[128:192], preferred_element_type=jnp.float32),
        0.0)


def _hr_matmul(a, b, c, w, blk=1000):
    n = a.shape[0]
    spec = pl.BlockSpec((blk, 64), lambda i: (i, 0))
    return pl.pallas_call(
        _hr_block,
        grid=(n // blk,),
        in_specs=[spec, spec, spec, pl.BlockSpec(w.shape, lambda i: (0, 0))],
        out_specs=pl.BlockSpec((blk, w.shape[1]), lambda i: (i, 0)),
        out_shape=jax.ShapeDtypeStruct((n, w.shape[1]), jnp.float32),
    )(a, b, c, w)


def _agg_block(f_ref, a_ref, b_ref, wa_ref, wb_ref, o_ref):
    h = jnp.maximum(
        jnp.dot(a_ref[...], wa_ref[...], preferred_element_type=jnp.float32)
        + jnp.dot(b_ref[...], wb_ref[...], preferred_element_type=jnp.float32),
        0.0)
    o_ref[...] = jnp.concatenate([f_ref[...], h], axis=1)


def _agg_matmul(feat, a, b, wa, wb, blk=1000):
    n = feat.shape[0]
    k = a.shape[1]
    return pl.pallas_call(
        _agg_block,
        grid=(n // blk,),
        in_specs=[pl.BlockSpec((blk, 64), lambda i: (i, 0)),
                  pl.BlockSpec((blk, k), lambda i: (i, 0)),
                  pl.BlockSpec((blk, k), lambda i: (i, 0)),
                  pl.BlockSpec(wa.shape, lambda i: (0, 0)),
                  pl.BlockSpec(wb.shape, lambda i: (0, 0))],
        out_specs=pl.BlockSpec((blk, 128), lambda i: (i, 0)),
        out_shape=jax.ShapeDtypeStruct((n, 128), jnp.float32),
    )(feat, a, b, wa, wb)


def _final_block(him_ref, hrm_ref, hum_ref, pe_ref, lab_ref, w_ref,
                 loss_ref, cnt_ref):
    him = him_ref[...]
    hrm = hrm_ref[...]
    hum = hum_ref[...]
    pe_raw = pe_ref[...]
    lab = lab_ref[...]
    w = w_ref[...]
    pe = jnp.maximum(pe_raw, 0.0)
    sq = jnp.sum(pe * pe, axis=1, keepdims=True)
    pe = pe * lax.rsqrt(jnp.maximum(sq, 1e-12))
    raw = (jnp.dot(him, w[0:128, :], preferred_element_type=jnp.float32)
           + jnp.dot(hrm, w[128:192, :], preferred_element_type=jnp.float32)
           + jnp.dot(hum, w[192:320, :], preferred_element_type=jnp.float32)
           + jnp.dot(pe, w[320:384, :], preferred_element_type=jnp.float32))
    mx = jnp.max(raw, axis=1, keepdims=True)
    e = jnp.exp(raw - mx)
    p = e / jnp.sum(e, axis=1, keepdims=True)
    z = lab * p
    loss = jnp.sum(jnp.log1p(jnp.exp(-z)))
    pred1 = raw[:, 1] > raw[:, 0]
    truth1 = lab[:, 1] > lab[:, 0]
    correct = jnp.sum((pred1 == truth1).astype(jnp.float32))
    loss_ref[...] = jnp.reshape(loss, (1, 1))
    cnt_ref[...] = jnp.reshape(correct, (1, 1))


def _final(him, hrm, hum, pe_raw, lab, w):
    return pl.pallas_call(
        _final_block,
        out_shape=(jax.ShapeDtypeStruct((1, 1), jnp.float32),
                   jax.ShapeDtypeStruct((1, 1), jnp.float32)),
    )(him, hrm, hum, pe_raw, lab, w)


def kernel(review_feat, user_feat, item_feat, r_feature, label,
           u_review_adj, u_item_adj, i_review_adj, i_user_adj,
           r_user_adj, r_item_adj, r_sup_src, r_sup_dst, r_sup_val,
           idx_mask, W_r_agg, W_user, W_item, W_gcn, u_weight):
    idx = idx_mask.astype(jnp.int32)

    adj2 = jnp.concatenate(
        [r_user_adj.astype(jnp.int32)[:, None],
         r_item_adj.astype(jnp.int32)[:, None],
         jnp.zeros((R, 14), jnp.int32)], axis=1)
    both = _sc_gather(adj2, idx)
    ru_idx, ri_idx = both[:, 0], both[:, 1]

    h_r_m = _hr_matmul(_sc_gather(review_feat, idx),
                       _sc_gather(user_feat, ru_idx),
                       _sc_gather(item_feat, ri_idx), W_r_agg)

    ur = _sc_gather(review_feat, u_review_adj.reshape(-1)).reshape(U, S * 64)
    riu = _sc_gather(item_feat, u_item_adj.reshape(-1)).reshape(U, S * 64)
    wu = W_user.reshape(S, 128, -1)
    h_u = _agg_matmul(user_feat, ur, riu,
                      wu[:, :64, :].reshape(S * 64, -1),
                      wu[:, 64:, :].reshape(S * 64, -1))
    ir = _sc_gather(review_feat, i_review_adj.reshape(-1)).reshape(I, S * 64)
    rui = _sc_gather(user_feat, i_user_adj.reshape(-1)).reshape(I, S * 64)
    wi = W_item.reshape(S, 128, -1)
    h_i = _agg_matmul(item_feat, ir, rui,
                      wi[:, :64, :].reshape(S * 64, -1),
                      wi[:, 64:, :].reshape(S * 64, -1))

    xw = r_feature @ W_gcn
    idx_pad = _pad_idx(idx, M_PAD)
    pos = _make_pos_build()(idx_pad)
    srcp = _pad_idx(r_sup_src, E_PAD)
    dstp = _pad_idx(r_sup_dst, E_PAD)
    valp = jnp.concatenate(
        [r_sup_val.astype(jnp.float32),
         jnp.zeros((E_PAD - r_sup_val.shape[0],), jnp.float32)])
    part0, part1 = _make_edge_pass()(xw, srcp, dstp, valp, pos)
    pe_m = _make_pe_assemble()(part0, part1, pos, idx_pad)[:M]

    him_g = _sc_gather(h_i, ri_idx)
    hum_g = _sc_gather(h_u, ru_idx)
    lab16 = jnp.concatenate([label, jnp.zeros((R, 14), jnp.float32)], axis=1)
    lab_m = _sc_gather(lab16, idx)[:, :CLASS]

    loss, cnt = _final(him_g, h_r_m, hum_g, pe_m, lab_m, u_weight)
    return (loss[0, 0], cnt[0, 0] / M)


# speedup vs baseline: 4.8984x; 1.3410x over previous
import functools

import jax
import jax.numpy as jnp
from jax import lax
from jax.experimental import pallas as pl
from jax.experimental.pallas import tpu as pltpu
from jax.experimental.pallas import tpu_sc as plsc

R = 50000
U = 10000
I = 5000
S = 7
M = 10000
CLASS = 2

NC = 2
NS = 16
NW = NC * NS


def _sc_mesh():
    return plsc.VectorSubcoreMesh(core_axis_name="c", subcore_axis_name="s")


@functools.cache
def _make_gather(B, D, dtype_name, CH):
    dtype = jnp.dtype(dtype_name)
    C = B // NW
    assert B % NW == 0 and C % CH == 0 and CH % 8 == 0
    nit = C // CH

    @functools.partial(
        pl.kernel,
        out_type=jax.ShapeDtypeStruct((B, D), dtype),
        mesh=_sc_mesh(),
        scratch_types=[pltpu.VMEM((CH,), jnp.int32),
                       pltpu.VMEM((CH, D), dtype),
                       pltpu.SemaphoreType.DMA],
        compiler_params=pltpu.CompilerParams(use_tc_tiling_on_sc=False,
                                              needs_layout_passes=False),
    )
    def g(table_hbm, idx_hbm, out_hbm, idx_v, rows_v, sem):
        wid = lax.axis_index("s") * NC + lax.axis_index("c")
        base = wid * C
        for j in range(nit):
            off = base + j * CH
            pltpu.sync_copy(idx_hbm.at[pl.ds(off, CH)], idx_v)
            pltpu.async_copy(table_hbm.at[idx_v], rows_v, sem).wait()
            pltpu.sync_copy(rows_v, out_hbm.at[pl.ds(off, CH)])

    return g


R_PAD = 50176
M_PAD = 10240
POS_C = R_PAD // NW
E_BLK = 512
E_W = 31744
E_PAD = E_W * NW
FIRE = 64
CBUF = 608
DUMMY = M


@functools.cache
def _make_pos_build():
    @functools.partial(
        pl.kernel,
        out_type=jax.ShapeDtypeStruct((R_PAD,), jnp.int32),
        mesh=_sc_mesh(),
        scratch_types=[pltpu.VMEM((M_PAD,), jnp.int32),
                       pltpu.VMEM((POS_C,), jnp.int32)],
        compiler_params=pltpu.CompilerParams(use_tc_tiling_on_sc=False,
                                              needs_layout_passes=False),
    )
    def pos_build(idx_hbm, pos_hbm, idx_v, pos_loc):
        wid = lax.axis_index("s") * NC + lax.axis_index("c")
        lo = wid * POS_C
        neg = jnp.full((16,), -1, jnp.int32)

        def fill(i, _):
            pos_loc[pl.ds(i * 16, 16)] = neg
            return 0
        lax.fori_loop(0, POS_C // 16, fill, 0)
        pltpu.sync_copy(idx_hbm, idx_v)
        iota = lax.iota(jnp.int32, 16)

        def scat(j, _):
            iv = idx_v[pl.ds(j * 16, 16)]
            mv = iota + j * 16
            msk = (iv >= lo) & (iv < lo + POS_C)
            plsc.store_scatter(pos_loc, [iv - lo], mv, mask=msk)
            return 0
        lax.fori_loop(0, M_PAD // 16, scat, 0)
        pltpu.sync_copy(pos_loc, pos_hbm.at[pl.ds(lo, POS_C)])

    return pos_build


@functools.cache
def _make_edge_pass():
    @functools.partial(
        pl.kernel,
        out_type=(jax.ShapeDtypeStruct((M_PAD, 64), jnp.float32),
                  jax.ShapeDtypeStruct((M_PAD, 64), jnp.float32)),
        mesh=_sc_mesh(),
        scratch_types=[pltpu.VMEM((R_PAD,), jnp.int32),
                       pltpu.VMEM((E_BLK,), jnp.int32),
                       pltpu.VMEM((E_BLK,), jnp.int32),
                       pltpu.VMEM((E_BLK,), jnp.float32),
                       pltpu.VMEM((CBUF,), jnp.int32),
                       pltpu.VMEM((CBUF,), jnp.int32),
                       pltpu.VMEM((CBUF,), jnp.float32),
                       pltpu.VMEM((FIRE,), jnp.int32),
                       pltpu.VMEM((FIRE,), jnp.int32),
                       pltpu.VMEM((FIRE, 64), jnp.float32),
                       pltpu.VMEM((128, 64), jnp.float32),
                       pltpu.VMEM_SHARED((M_PAD, 64), jnp.float32),
                       pltpu.SemaphoreType.DMA,
                       pltpu.SemaphoreType.DMA,
                       pltpu.SemaphoreType.DMA],
        compiler_params=pltpu.CompilerParams(use_tc_tiling_on_sc=False,
                                              needs_layout_passes=False),
    )
    def edge_pass(xw_hbm, src_hbm, dst_hbm, val_hbm, pos_hbm, out0, out1,
                  pos_tile, src_ch, dst_ch, val_ch, srcC, posC, valC,
                  srcF, posF, rows, zbuf, acc, sem, sem2, sem3):
        cid = lax.axis_index("c")
        sid = lax.axis_index("s")
        wid = sid * NC + cid
        rows_per_sub = M_PAD // NS
        pltpu.sync_copy(pos_hbm, pos_tile)
        zero16 = jnp.zeros((16,), jnp.float32)
        z16i = jnp.zeros((16,), jnp.int32)
        d16 = jnp.full((16,), DUMMY, jnp.int32)
        iota = lax.iota(jnp.int32, 16)

        def zf(i, _):
            for c in range(4):
                zbuf[i, pl.ds(c * 16, 16)] = zero16
            return 0
        lax.fori_loop(0, 128, zf, 0)
        for t in range(rows_per_sub // 128):
            pltpu.sync_copy(zbuf,
                            acc.at[pl.ds(sid * rows_per_sub + t * 128, 128)])

        def pf(j, _):
            srcC[pl.ds(j * 16, 16)] = z16i
            posC[pl.ds(j * 16, 16)] = d16
            valC[pl.ds(j * 16, 16)] = zero16
            return 0
        lax.fori_loop(0, CBUF // 16, pf, 0)
        plsc.subcore_barrier()

        base = wid * E_W

        def blk(b, _):
            off = base + b * E_BLK
            c1 = pltpu.async_copy(src_hbm.at[pl.ds(off, E_BLK)], src_ch, sem)
            c2 = pltpu.async_copy(dst_hbm.at[pl.ds(off, E_BLK)], dst_ch, sem2)
            c3 = pltpu.async_copy(val_hbm.at[pl.ds(off, E_BLK)], val_ch, sem3)
            c1.wait()
            c2.wait()
            c3.wait()

            def cmp(g, k):
                dv = dst_ch[pl.ds(g * 16, 16)]
                pv = plsc.load_gather(pos_tile, [dv])
                msk = pv >= 0
                sv = src_ch[pl.ds(g * 16, 16)]
                vv = val_ch[pl.ds(g * 16, 16)]
                plsc.store_compressed(srcC.at[pl.ds(k, 16)], sv, mask=msk)
                plsc.store_compressed(posC.at[pl.ds(k, 16)], pv, mask=msk)
                plsc.store_compressed(valC.at[pl.ds(k, 16)], vv, mask=msk)
                return k + jnp.sum(msk.astype(jnp.int32))
            k = lax.fori_loop(0, E_BLK // 16, cmp, jnp.int32(0))

            p0 = (k // 16) * 16
            for t in range(5):
                ids = iota + (p0 + t * 16)
                m2 = ids >= k
                plsc.store_scatter(srcC, [ids], z16i, mask=m2)
                plsc.store_scatter(posC, [ids], d16, mask=m2)
                plsc.store_scatter(valC, [ids], zero16, mask=m2)

            nf = (k + FIRE - 1) // FIRE

            def fire(t, _):
                o = t * FIRE
                for u in range(FIRE // 16):
                    srcF[pl.ds(u * 16, 16)] = srcC[pl.ds(o + u * 16, 16)]
                    posF[pl.ds(u * 16, 16)] = posC[pl.ds(o + u * 16, 16)]
                pltpu.async_copy(xw_hbm.at[srcF], rows, sem).wait()
                for g in range(FIRE // 16):
                    valv = valC[pl.ds(o + g * 16, 16)]
                    for j in range(16):
                        e = g * 16 + j
                        v = valv[j]
                        for c in range(4):
                            s = pl.ds(c * 16, 16)
                            rows[e, s] = rows[e, s] * v
                pltpu.sync_copy(rows, acc.at[posF], add=True)
                return 0
            lax.fori_loop(0, nf, fire, 0)
            return 0
        lax.fori_loop(0, E_W // E_BLK, blk, 0)

        plsc.subcore_barrier()
        for t in range(rows_per_sub // 128):
            o = sid * rows_per_sub + t * 128
            pltpu.sync_copy(acc.at[pl.ds(o, 128)], zbuf)

            @pl.when(cid == 0)
            def _():
                pltpu.sync_copy(zbuf, out0.at[pl.ds(o, 128)])

            @pl.when(cid == 1)
            def _():
                pltpu.sync_copy(zbuf, out1.at[pl.ds(o, 128)])

    return edge_pass


@functools.cache
def _make_pe_assemble():
    C = M_PAD // NW

    @functools.partial(
        pl.kernel,
        out_type=jax.ShapeDtypeStruct((M_PAD, 64), jnp.float32),
        mesh=_sc_mesh(),
        scratch_types=[pltpu.VMEM((R_PAD,), jnp.int32),
                       pltpu.VMEM((C,), jnp.int32),
                       pltpu.VMEM((C,), jnp.int32),
                       pltpu.VMEM((C, 64), jnp.float32),
                       pltpu.VMEM((C, 64), jnp.float32),
                       pltpu.SemaphoreType.DMA],
        compiler_params=pltpu.CompilerParams(use_tc_tiling_on_sc=False,
                                              needs_layout_passes=False),
    )
    def pe_assemble(part0, part1, pos_hbm, idx_hbm, out,
                    pos_tile, idx_v, w_v, r0, r1, sem):
        wid = lax.axis_index("s") * NC + lax.axis_index("c")
        base = wid * C
        pltpu.sync_copy(pos_hbm, pos_tile)
        pltpu.sync_copy(idx_hbm.at[pl.ds(base, C)], idx_v)

        def wg(j, _):
            iv = idx_v[pl.ds(j * 16, 16)]
            w_v[pl.ds(j * 16, 16)] = plsc.load_gather(pos_tile, [iv])
            return 0
        lax.fori_loop(0, C // 16, wg, 0)
        pltpu.async_copy(part0.at[w_v], r0, sem).wait()
        pltpu.async_copy(part1.at[w_v], r1, sem).wait()

        def addr(i, _):
            for c in range(4):
                s = pl.ds(c * 16, 16)
                r0[i, s] = r0[i, s] + r1[i, s]
            return 0
        lax.fori_loop(0, C, addr, 0)
        pltpu.sync_copy(r0, out.at[pl.ds(base, C)])

    return pe_assemble


def _pad_idx(idx, B):
    return jnp.concatenate(
        [idx.astype(jnp.int32),
         jnp.zeros((B - idx.shape[0],), jnp.int32)])


def _sc_gather(table, idx, CH=320):
    n = idx.shape[0]
    B = -(-n // (NW * CH)) * (NW * CH)
    g = _make_gather(B, table.shape[1], str(table.dtype), CH)
    return g(table, _pad_idx(idx, B))[:n]


def _xw_block(x_ref, w_ref, o_ref):
    o_ref[...] = jnp.dot(x_ref[...], w_ref[...],
                         preferred_element_type=jnp.float32)


def _xw_matmul(x, w, blk=5000):
    n = x.shape[0]
    return pl.pallas_call(
        _xw_block,
        grid=(n // blk,),
        in_specs=[pl.BlockSpec((blk, x.shape[1]), lambda i: (i, 0)),
                  pl.BlockSpec(w.shape, lambda i: (0, 0))],
        out_specs=pl.BlockSpec((blk, w.shape[1]), lambda i: (i, 0)),
        out_shape=jax.ShapeDtypeStruct((n, w.shape[1]), jnp.float32),
    )(x, w)


def _hr_block(a_ref, b_ref, c_ref, w_ref, o_ref):
    w = w_ref[...]
    o_ref[...] = jnp.maximum(
        jnp.dot(a_ref[...], w[0:64], preferred_element_type=jnp.float32)
        + jnp.dot(b_ref[...], w[64:128], preferred_element_type=jnp.float32)
        + jnp.dot(c_ref[...], w[128:192], preferred_element_type=jnp.float32),
        0.0)


def _hr_matmul(a, b, c, w, blk=1000):
    n = a.shape[0]
    spec = pl.BlockSpec((blk, 64), lambda i: (i, 0))
    return pl.pallas_call(
        _hr_block,
        grid=(n // blk,),
        in_specs=[spec, spec, spec, pl.BlockSpec(w.shape, lambda i: (0, 0))],
        out_specs=pl.BlockSpec((blk, w.shape[1]), lambda i: (i, 0)),
        out_shape=jax.ShapeDtypeStruct((n, w.shape[1]), jnp.float32),
    )(a, b, c, w)


def _agg_block(f_ref, a_ref, b_ref, wa_ref, wb_ref, o_ref):
    h = jnp.maximum(
        jnp.dot(a_ref[...], wa_ref[...], preferred_element_type=jnp.float32)
        + jnp.dot(b_ref[...], wb_ref[...], preferred_element_type=jnp.float32),
        0.0)
    o_ref[...] = jnp.concatenate([f_ref[...], h], axis=1)


def _agg_matmul(feat, a, b, wa, wb, blk=1000):
    n = feat.shape[0]
    k = a.shape[1]
    return pl.pallas_call(
        _agg_block,
        grid=(n // blk,),
        in_specs=[pl.BlockSpec((blk, 64), lambda i: (i, 0)),
                  pl.BlockSpec((blk, k), lambda i: (i, 0)),
                  pl.BlockSpec((blk, k), lambda i: (i, 0)),
                  pl.BlockSpec(wa.shape, lambda i: (0, 0)),
                  pl.BlockSpec(wb.shape, lambda i: (0, 0))],
        out_specs=pl.BlockSpec((blk, 128), lambda i: (i, 0)),
        out_shape=jax.ShapeDtypeStruct((n, 128), jnp.float32),
    )(feat, a, b, wa, wb)


def _final_block(him_ref, hrm_ref, hum_ref, pe_ref, lab_ref, w_ref,
                 loss_ref, cnt_ref):
    him = him_ref[...]
    hrm = hrm_ref[...]
    hum = hum_ref[...]
    pe_raw = pe_ref[...]
    lab = lab_ref[...]
    w = w_ref[...]
    pe = jnp.maximum(pe_raw, 0.0)
    sq = jnp.sum(pe * pe, axis=1, keepdims=True)
    pe = pe * lax.rsqrt(jnp.maximum(sq, 1e-12))
    raw = (jnp.dot(him, w[0:128, :], preferred_element_type=jnp.float32)
           + jnp.dot(hrm, w[128:192, :], preferred_element_type=jnp.float32)
           + jnp.dot(hum, w[192:320, :], preferred_element_type=jnp.float32)
           + jnp.dot(pe, w[320:384, :], preferred_element_type=jnp.float32))
    mx = jnp.max(raw, axis=1, keepdims=True)
    e = jnp.exp(raw - mx)
    p = e / jnp.sum(e, axis=1, keepdims=True)
    z = lab * p
    loss = jnp.sum(jnp.log1p(jnp.exp(-z)))
    pred1 = raw[:, 1] > raw[:, 0]
    truth1 = lab[:, 1] > lab[:, 0]
    correct = jnp.sum((pred1 == truth1).astype(jnp.float32))
    loss_ref[...] = jnp.reshape(loss, (1, 1))
    cnt_ref[...] = jnp.reshape(correct, (1, 1))


def _final(him, hrm, hum, pe_raw, lab, w):
    return pl.pallas_call(
        _final_block,
        out_shape=(jax.ShapeDtypeStruct((1, 1), jnp.float32),
                   jax.ShapeDtypeStruct((1, 1), jnp.float32)),
    )(him, hrm, hum, pe_raw, lab, w)


def kernel(review_feat, user_feat, item_feat, r_feature, label,
           u_review_adj, u_item_adj, i_review_adj, i_user_adj,
           r_user_adj, r_item_adj, r_sup_src, r_sup_dst, r_sup_val,
           idx_mask, W_r_agg, W_user, W_item, W_gcn, u_weight):
    idx = idx_mask.astype(jnp.int32)

    adj2 = jnp.concatenate(
        [r_user_adj.astype(jnp.int32)[:, None],
         r_item_adj.astype(jnp.int32)[:, None],
         jnp.zeros((R, 14), jnp.int32)], axis=1)
    both = _sc_gather(adj2, idx)
    ru_idx, ri_idx = both[:, 0], both[:, 1]

    h_r_m = _hr_matmul(_sc_gather(review_feat, idx),
                       _sc_gather(user_feat, ru_idx),
                       _sc_gather(item_feat, ri_idx), W_r_agg)

    ur = _sc_gather(review_feat, u_review_adj.reshape(-1)).reshape(U, S * 64)
    riu = _sc_gather(item_feat, u_item_adj.reshape(-1)).reshape(U, S * 64)
    wu = W_user.reshape(S, 128, -1)
    h_u = _agg_matmul(user_feat, ur, riu,
                      wu[:, :64, :].reshape(S * 64, -1),
                      wu[:, 64:, :].reshape(S * 64, -1))
    ir = _sc_gather(review_feat, i_review_adj.reshape(-1)).reshape(I, S * 64)
    rui = _sc_gather(user_feat, i_user_adj.reshape(-1)).reshape(I, S * 64)
    wi = W_item.reshape(S, 128, -1)
    h_i = _agg_matmul(item_feat, ir, rui,
                      wi[:, :64, :].reshape(S * 64, -1),
                      wi[:, 64:, :].reshape(S * 64, -1))

    xw = r_feature @ W_gcn
    idx_pad = _pad_idx(idx, M_PAD)
    pos = _make_pos_build()(idx_pad)
    srcp = _pad_idx(r_sup_src, E_PAD)
    dstp = _pad_idx(r_sup_dst, E_PAD)
    valp = jnp.concatenate(
        [r_sup_val.astype(jnp.float32),
         jnp.zeros((E_PAD - r_sup_val.shape[0],), jnp.float32)])
    part0, part1 = _make_edge_pass()(xw, srcp, dstp, valp, pos)
    pe_m = _make_pe_assemble()(part0, part1, pos, idx_pad)[:M]

    him_g = _sc_gather(h_i, ri_idx)
    hum_g = _sc_gather(h_u, ru_idx)
    lab16 = jnp.concatenate([label, jnp.zeros((R, 14), jnp.float32)], axis=1)
    lab_m = _sc_gather(lab16, idx)[:, :CLASS]

    loss, cnt = _final(him_g, h_r_m, hum_g, pe_m, lab_m, u_weight)
    return (loss[0, 0], cnt[0, 0] / M)


# speedup vs baseline: 4.9970x; 1.0201x over previous
import functools

import jax
import jax.numpy as jnp
from jax import lax
from jax.experimental import pallas as pl
from jax.experimental.pallas import tpu as pltpu
from jax.experimental.pallas import tpu_sc as plsc

R = 50000
U = 10000
I = 5000
S = 7
M = 10000
CLASS = 2

NC = 2
NS = 16
NW = NC * NS


def _sc_mesh():
    return plsc.VectorSubcoreMesh(core_axis_name="c", subcore_axis_name="s")


@functools.cache
def _make_gather(B, D, dtype_name, CH):
    dtype = jnp.dtype(dtype_name)
    C = B // NW
    assert B % NW == 0 and C % CH == 0 and CH % 8 == 0
    nit = C // CH

    @functools.partial(
        pl.kernel,
        out_type=jax.ShapeDtypeStruct((B, D), dtype),
        mesh=_sc_mesh(),
        scratch_types=[pltpu.VMEM((CH,), jnp.int32),
                       pltpu.VMEM((CH, D), dtype),
                       pltpu.SemaphoreType.DMA],
        compiler_params=pltpu.CompilerParams(use_tc_tiling_on_sc=False,
                                              needs_layout_passes=False),
    )
    def g(table_hbm, idx_hbm, out_hbm, idx_v, rows_v, sem):
        wid = lax.axis_index("s") * NC + lax.axis_index("c")
        base = wid * C
        for j in range(nit):
            off = base + j * CH
            pltpu.sync_copy(idx_hbm.at[pl.ds(off, CH)], idx_v)
            pltpu.async_copy(table_hbm.at[idx_v], rows_v, sem).wait()
            pltpu.sync_copy(rows_v, out_hbm.at[pl.ds(off, CH)])

    return g


R_PAD = 50176
M_PAD = 10240
POS_C = R_PAD // NW
E_BLK = 1024
E_W = 31744
E_PAD = E_W * NW
FIRE = 128
CBUF = 1184
DUMMY = M


@functools.cache
def _make_pos_build():
    @functools.partial(
        pl.kernel,
        out_type=jax.ShapeDtypeStruct((R_PAD,), jnp.int32),
        mesh=_sc_mesh(),
        scratch_types=[pltpu.VMEM((M_PAD,), jnp.int32),
                       pltpu.VMEM((POS_C,), jnp.int32)],
        compiler_params=pltpu.CompilerParams(use_tc_tiling_on_sc=False,
                                              needs_layout_passes=False),
    )
    def pos_build(idx_hbm, pos_hbm, idx_v, pos_loc):
        wid = lax.axis_index("s") * NC + lax.axis_index("c")
        lo = wid * POS_C
        neg = jnp.full((16,), -1, jnp.int32)

        def fill(i, _):
            pos_loc[pl.ds(i * 16, 16)] = neg
            return 0
        lax.fori_loop(0, POS_C // 16, fill, 0)
        pltpu.sync_copy(idx_hbm, idx_v)
        iota = lax.iota(jnp.int32, 16)

        def scat(j, _):
            iv = idx_v[pl.ds(j * 16, 16)]
            mv = iota + j * 16
            msk = (iv >= lo) & (iv < lo + POS_C)
            plsc.store_scatter(pos_loc, [iv - lo], mv, mask=msk)
            return 0
        lax.fori_loop(0, M_PAD // 16, scat, 0)
        pltpu.sync_copy(pos_loc, pos_hbm.at[pl.ds(lo, POS_C)])

    return pos_build


@functools.cache
def _make_edge_pass():
    @functools.partial(
        pl.kernel,
        out_type=(jax.ShapeDtypeStruct((M_PAD, 64), jnp.float32),
                  jax.ShapeDtypeStruct((M_PAD, 64), jnp.float32)),
        mesh=_sc_mesh(),
        scratch_types=[pltpu.VMEM((R_PAD,), jnp.int32),
                       pltpu.VMEM((E_BLK,), jnp.int32),
                       pltpu.VMEM((E_BLK,), jnp.int32),
                       pltpu.VMEM((E_BLK,), jnp.float32),
                       pltpu.VMEM((CBUF,), jnp.int32),
                       pltpu.VMEM((CBUF,), jnp.int32),
                       pltpu.VMEM((CBUF,), jnp.float32),
                       pltpu.VMEM((FIRE,), jnp.int32),
                       pltpu.VMEM((FIRE,), jnp.int32),
                       pltpu.VMEM((FIRE, 64), jnp.float32),
                       pltpu.VMEM((128, 64), jnp.float32),
                       pltpu.VMEM_SHARED((M_PAD, 64), jnp.float32),
                       pltpu.SemaphoreType.DMA,
                       pltpu.SemaphoreType.DMA,
                       pltpu.SemaphoreType.DMA],
        compiler_params=pltpu.CompilerParams(use_tc_tiling_on_sc=False,
                                              needs_layout_passes=False),
    )
    def edge_pass(xw_hbm, src_hbm, dst_hbm, val_hbm, pos_hbm, out0, out1,
                  pos_tile, src_ch, dst_ch, val_ch, srcC, posC, valC,
                  srcF, posF, rows, zbuf, acc, sem, sem2, sem3):
        cid = lax.axis_index("c")
        sid = lax.axis_index("s")
        wid = sid * NC + cid
        rows_per_sub = M_PAD // NS
        pltpu.sync_copy(pos_hbm, pos_tile)
        zero16 = jnp.zeros((16,), jnp.float32)
        z16i = jnp.zeros((16,), jnp.int32)
        d16 = jnp.full((16,), DUMMY, jnp.int32)
        iota = lax.iota(jnp.int32, 16)

        def zf(i, _):
            for c in range(4):
                zbuf[i, pl.ds(c * 16, 16)] = zero16
            return 0
        lax.fori_loop(0, 128, zf, 0)
        for t in range(rows_per_sub // 128):
            pltpu.sync_copy(zbuf,
                            acc.at[pl.ds(sid * rows_per_sub + t * 128, 128)])

        def pf(j, _):
            srcC[pl.ds(j * 16, 16)] = z16i
            posC[pl.ds(j * 16, 16)] = d16
            valC[pl.ds(j * 16, 16)] = zero16
            return 0
        lax.fori_loop(0, CBUF // 16, pf, 0)
        plsc.subcore_barrier()

        base = wid * E_W

        def blk(b, _):
            off = base + b * E_BLK
            c1 = pltpu.async_copy(src_hbm.at[pl.ds(off, E_BLK)], src_ch, sem)
            c2 = pltpu.async_copy(dst_hbm.at[pl.ds(off, E_BLK)], dst_ch, sem2)
            c3 = pltpu.async_copy(val_hbm.at[pl.ds(off, E_BLK)], val_ch, sem3)
            c1.wait()
            c2.wait()
            c3.wait()

            def cmp(g, k):
                dv = dst_ch[pl.ds(g * 16, 16)]
                pv = plsc.load_gather(pos_tile, [dv])
                msk = pv >= 0
                sv = src_ch[pl.ds(g * 16, 16)]
                vv = val_ch[pl.ds(g * 16, 16)]
                plsc.store_compressed(srcC.at[pl.ds(k, 16)], sv, mask=msk)
                plsc.store_compressed(posC.at[pl.ds(k, 16)], pv, mask=msk)
                plsc.store_compressed(valC.at[pl.ds(k, 16)], vv, mask=msk)
                return k + jnp.sum(msk.astype(jnp.int32))
            k = lax.fori_loop(0, E_BLK // 16, cmp, jnp.int32(0))

            p0 = (k // 16) * 16
            for t in range(FIRE // 16 + 1):
                ids = iota + (p0 + t * 16)
                m2 = ids >= k
                plsc.store_scatter(srcC, [ids], z16i, mask=m2)
                plsc.store_scatter(posC, [ids], d16, mask=m2)
                plsc.store_scatter(valC, [ids], zero16, mask=m2)

            nf = (k + FIRE - 1) // FIRE

            def fire(t, _):
                o = t * FIRE
                for u in range(FIRE // 16):
                    srcF[pl.ds(u * 16, 16)] = srcC[pl.ds(o + u * 16, 16)]
                    posF[pl.ds(u * 16, 16)] = posC[pl.ds(o + u * 16, 16)]
                pltpu.async_copy(xw_hbm.at[srcF], rows, sem).wait()
                for g in range(FIRE // 16):
                    valv = valC[pl.ds(o + g * 16, 16)]
                    for j in range(16):
                        e = g * 16 + j
                        v = valv[j]
                        for c in range(4):
                            s = pl.ds(c * 16, 16)
                            rows[e, s] = rows[e, s] * v
                pltpu.sync_copy(rows, acc.at[posF], add=True)
                return 0
            lax.fori_loop(0, nf, fire, 0)
            return 0
        lax.fori_loop(0, E_W // E_BLK, blk, 0)

        plsc.subcore_barrier()
        for t in range(rows_per_sub // 128):
            o = sid * rows_per_sub + t * 128
            pltpu.sync_copy(acc.at[pl.ds(o, 128)], zbuf)

            @pl.when(cid == 0)
            def _():
                pltpu.sync_copy(zbuf, out0.at[pl.ds(o, 128)])

            @pl.when(cid == 1)
            def _():
                pltpu.sync_copy(zbuf, out1.at[pl.ds(o, 128)])

    return edge_pass


@functools.cache
def _make_pe_assemble():
    C = M_PAD // NW

    @functools.partial(
        pl.kernel,
        out_type=jax.ShapeDtypeStruct((M_PAD, 64), jnp.float32),
        mesh=_sc_mesh(),
        scratch_types=[pltpu.VMEM((R_PAD,), jnp.int32),
                       pltpu.VMEM((C,), jnp.int32),
                       pltpu.VMEM((C,), jnp.int32),
                       pltpu.VMEM((C, 64), jnp.float32),
                       pltpu.VMEM((C, 64), jnp.float32),
                       pltpu.SemaphoreType.DMA],
        compiler_params=pltpu.CompilerParams(use_tc_tiling_on_sc=False,
                                              needs_layout_passes=False),
    )
    def pe_assemble(part0, part1, pos_hbm, idx_hbm, out,
                    pos_tile, idx_v, w_v, r0, r1, sem):
        wid = lax.axis_index("s") * NC + lax.axis_index("c")
        base = wid * C
        pltpu.sync_copy(pos_hbm, pos_tile)
        pltpu.sync_copy(idx_hbm.at[pl.ds(base, C)], idx_v)

        def wg(j, _):
            iv = idx_v[pl.ds(j * 16, 16)]
            w_v[pl.ds(j * 16, 16)] = plsc.load_gather(pos_tile, [iv])
            return 0
        lax.fori_loop(0, C // 16, wg, 0)
        pltpu.async_copy(part0.at[w_v], r0, sem).wait()
        pltpu.async_copy(part1.at[w_v], r1, sem).wait()

        def addr(i, _):
            for c in range(4):
                s = pl.ds(c * 16, 16)
                r0[i, s] = r0[i, s] + r1[i, s]
            return 0
        lax.fori_loop(0, C, addr, 0)
        pltpu.sync_copy(r0, out.at[pl.ds(base, C)])

    return pe_assemble


def _pad_idx(idx, B):
    return jnp.concatenate(
        [idx.astype(jnp.int32),
         jnp.zeros((B - idx.shape[0],), jnp.int32)])


def _sc_gather(table, idx, CH=320):
    n = idx.shape[0]
    B = -(-n // (NW * CH)) * (NW * CH)
    g = _make_gather(B, table.shape[1], str(table.dtype), CH)
    return g(table, _pad_idx(idx, B))[:n]


def _xw_block(x_ref, w_ref, o_ref):
    o_ref[...] = jnp.dot(x_ref[...], w_ref[...],
                         preferred_element_type=jnp.float32)


def _xw_matmul(x, w, blk=5000):
    n = x.shape[0]
    return pl.pallas_call(
        _xw_block,
        grid=(n // blk,),
        in_specs=[pl.BlockSpec((blk, x.shape[1]), lambda i: (i, 0)),
                  pl.BlockSpec(w.shape, lambda i: (0, 0))],
        out_specs=pl.BlockSpec((blk, w.shape[1]), lambda i: (i, 0)),
        out_shape=jax.ShapeDtypeStruct((n, w.shape[1]), jnp.float32),
    )(x, w)


def _hr_block(a_ref, b_ref, c_ref, w_ref, o_ref):
    w = w_ref[...]
    o_ref[...] = jnp.maximum(
        jnp.dot(a_ref[...], w[0:64], preferred_element_type=jnp.float32)
        + jnp.dot(b_ref[...], w[64:128], preferred_element_type=jnp.float32)
        + jnp.dot(c_ref[...], w[128:192], preferred_element_type=jnp.float32),
        0.0)


def _hr_matmul(a, b, c, w, blk=1000):
    n = a.shape[0]
    spec = pl.BlockSpec((blk, 64), lambda i: (i, 0))
    return pl.pallas_call(
        _hr_block,
        grid=(n // blk,),
        in_specs=[spec, spec, spec, pl.BlockSpec(w.shape, lambda i: (0, 0))],
        out_specs=pl.BlockSpec((blk, w.shape[1]), lambda i: (i, 0)),
        out_shape=jax.ShapeDtypeStruct((n, w.shape[1]), jnp.float32),
    )(a, b, c, w)


def _agg_block(f_ref, a_ref, b_ref, wa_ref, wb_ref, o_ref):
    h = jnp.maximum(
        jnp.dot(a_ref[...], wa_ref[...], preferred_element_type=jnp.float32)
        + jnp.dot(b_ref[...], wb_ref[...], preferred_element_type=jnp.float32),
        0.0)
    o_ref[...] = jnp.concatenate([f_ref[...], h], axis=1)


def _agg_matmul(feat, a, b, wa, wb, blk=1000):
    n = feat.shape[0]
    k = a.shape[1]
    return pl.pallas_call(
        _agg_block,
        grid=(n // blk,),
        in_specs=[pl.BlockSpec((blk, 64), lambda i: (i, 0)),
                  pl.BlockSpec((blk, k), lambda i: (i, 0)),
                  pl.BlockSpec((blk, k), lambda i: (i, 0)),
                  pl.BlockSpec(wa.shape, lambda i: (0, 0)),
                  pl.BlockSpec(wb.shape, lambda i: (0, 0))],
        out_specs=pl.BlockSpec((blk, 128), lambda i: (i, 0)),
        out_shape=jax.ShapeDtypeStruct((n, 128), jnp.float32),
    )(feat, a, b, wa, wb)


def _final_block(him_ref, hrm_ref, hum_ref, pe_ref, lab_ref, w_ref,
                 loss_ref, cnt_ref):
    him = him_ref[...]
    hrm = hrm_ref[...]
    hum = hum_ref[...]
    pe_raw = pe_ref[...]
    lab = lab_ref[...]
    w = w_ref[...]
    pe = jnp.maximum(pe_raw, 0.0)
    sq = jnp.sum(pe * pe, axis=1, keepdims=True)
    pe = pe * lax.rsqrt(jnp.maximum(sq, 1e-12))
    raw = (jnp.dot(him, w[0:128, :], preferred_element_type=jnp.float32)
           + jnp.dot(hrm, w[128:192, :], preferred_element_type=jnp.float32)
           + jnp.dot(hum, w[192:320, :], preferred_element_type=jnp.float32)
           + jnp.dot(pe, w[320:384, :], preferred_element_type=jnp.float32))
    mx = jnp.max(raw, axis=1, keepdims=True)
    e = jnp.exp(raw - mx)
    p = e / jnp.sum(e, axis=1, keepdims=True)
    z = lab * p
    loss = jnp.sum(jnp.log1p(jnp.exp(-z)))
    pred1 = raw[:, 1] > raw[:, 0]
    truth1 = lab[:, 1] > lab[:, 0]
    correct = jnp.sum((pred1 == truth1).astype(jnp.float32))
    loss_ref[...] = jnp.reshape(loss, (1, 1))
    cnt_ref[...] = jnp.reshape(correct, (1, 1))


def _final(him, hrm, hum, pe_raw, lab, w):
    return pl.pallas_call(
        _final_block,
        out_shape=(jax.ShapeDtypeStruct((1, 1), jnp.float32),
                   jax.ShapeDtypeStruct((1, 1), jnp.float32)),
    )(him, hrm, hum, pe_raw, lab, w)


def kernel(review_feat, user_feat, item_feat, r_feature, label,
           u_review_adj, u_item_adj, i_review_adj, i_user_adj,
           r_user_adj, r_item_adj, r_sup_src, r_sup_dst, r_sup_val,
           idx_mask, W_r_agg, W_user, W_item, W_gcn, u_weight):
    idx = idx_mask.astype(jnp.int32)

    adj2 = jnp.concatenate(
        [r_user_adj.astype(jnp.int32)[:, None],
         r_item_adj.astype(jnp.int32)[:, None],
         jnp.zeros((R, 14), jnp.int32)], axis=1)
    both = _sc_gather(adj2, idx)
    ru_idx, ri_idx = both[:, 0], both[:, 1]

    h_r_m = _hr_matmul(_sc_gather(review_feat, idx),
                       _sc_gather(user_feat, ru_idx),
                       _sc_gather(item_feat, ri_idx), W_r_agg)

    ur = _sc_gather(review_feat, u_review_adj.reshape(-1)).reshape(U, S * 64)
    riu = _sc_gather(item_feat, u_item_adj.reshape(-1)).reshape(U, S * 64)
    wu = W_user.reshape(S, 128, -1)
    h_u = _agg_matmul(user_feat, ur, riu,
                      wu[:, :64, :].reshape(S * 64, -1),
                      wu[:, 64:, :].reshape(S * 64, -1))
    ir = _sc_gather(review_feat, i_review_adj.reshape(-1)).reshape(I, S * 64)
    rui = _sc_gather(user_feat, i_user_adj.reshape(-1)).reshape(I, S * 64)
    wi = W_item.reshape(S, 128, -1)
    h_i = _agg_matmul(item_feat, ir, rui,
                      wi[:, :64, :].reshape(S * 64, -1),
                      wi[:, 64:, :].reshape(S * 64, -1))

    xw = r_feature @ W_gcn
    idx_pad = _pad_idx(idx, M_PAD)
    pos = _make_pos_build()(idx_pad)
    srcp = _pad_idx(r_sup_src, E_PAD)
    dstp = _pad_idx(r_sup_dst, E_PAD)
    valp = jnp.concatenate(
        [r_sup_val.astype(jnp.float32),
         jnp.zeros((E_PAD - r_sup_val.shape[0],), jnp.float32)])
    part0, part1 = _make_edge_pass()(xw, srcp, dstp, valp, pos)
    pe_m = _make_pe_assemble()(part0, part1, pos, idx_pad)[:M]

    him_g = _sc_gather(h_i, ri_idx)
    hum_g = _sc_gather(h_u, ru_idx)
    lab16 = jnp.concatenate([label, jnp.zeros((R, 14), jnp.float32)], axis=1)
    lab_m = _sc_gather(lab16, idx)[:, :CLASS]

    loss, cnt = _final(him_g, h_r_m, hum_g, pe_m, lab_m, u_weight)
    return (loss[0, 0], cnt[0, 0] / M)
